# baseline (speedup 1.0000x reference)
import functools

import jax
import jax.numpy as jnp
import numpy as np
from jax import lax
from jax.experimental import pallas as pl
from jax.experimental.pallas import tpu as pltpu
from jax.experimental.pallas import tpu_sc as plsc

_B, _S, _D = 4096, 200, 64
_NSUB = _B * 25
_NC, _NS = 2, 16
_NW = _NC * _NS
_B_PER_WORKER = _B // _NW


def _tables():
    key = jax.random.key(42)
    k_mask, k_perm = jax.random.split(key)
    mask = np.asarray(jax.random.bernoulli(k_mask, 0.1, (_B, _S, _D)))
    perm = np.asarray(jax.random.permutation(k_perm, _B)).astype(np.int32)
    m = mask.reshape(_NSUB, 32, 16).astype(np.uint32)
    packed = (m << np.arange(32, dtype=np.uint32)[None, :, None]).sum(
        axis=1, dtype=np.uint32)
    pm = np.zeros((_B, 1, 512), dtype=np.uint32)
    pm[:, 0, :400] = packed.reshape(_B, 400)
    return pm, perm.reshape(_B, 1)


_PMASK, _PERM = _tables()


@functools.cache
def _make_swap_kernel():
    mesh = plsc.VectorSubcoreMesh(core_axis_name="c", subcore_axis_name="s")
    nbuf = 2

    @functools.partial(
        pl.kernel,
        mesh=mesh,
        out_type=jax.ShapeDtypeStruct((_B, _S, _D), jnp.float32),
        compiler_params=pltpu.CompilerParams(use_tc_tiling_on_sc=False),
        scratch_types=[
            pltpu.VMEM((_B_PER_WORKER, 1), jnp.int32),
            *[pltpu.VMEM((_S, _D), jnp.float32) for _ in range(nbuf)],
            *[pltpu.VMEM((1, _S, _D), jnp.float32) for _ in range(nbuf)],
            *[pltpu.VMEM((_S, _D), jnp.float32) for _ in range(nbuf)],
            *[pltpu.VMEM((1, 512), jnp.uint32) for _ in range(nbuf)],
            *[pltpu.SemaphoreType.DMA for _ in range(3 * nbuf)],
        ],
    )
    def _swap_kernel(x_hbm, pm_hbm, perm_hbm, out_hbm, idx_v,
                     x_v0, x_v1, swap_v0, swap_v1, out_v0, out_v1,
                     p_v0, p_v1, in_s0, in_s1, g_s0, g_s1, o_s0, o_s1):
        x_vs, swap_vs, out_vs, p_vs = ((x_v0, x_v1), (swap_v0, swap_v1),
                                       (out_v0, out_v1), (p_v0, p_v1))
        in_sems, g_sems, o_sems = (in_s0, in_s1), (g_s0, g_s1), (o_s0, o_s1)
        wid = lax.axis_index("s") * _NC + lax.axis_index("c")
        bbase = wid * _B_PER_WORKER
        pltpu.sync_copy(perm_hbm.at[pl.ds(bbase, _B_PER_WORKER)], idx_v)

        def issue_reads(b, gg):
            row = bbase + gg
            pltpu.async_copy(x_hbm.at[row], x_vs[b], in_sems[b])
            pltpu.async_copy(pm_hbm.at[row], p_vs[b], in_sems[b])
            pltpu.async_copy(x_hbm.at[idx_v.at[gg]], swap_vs[b], g_sems[b])

        def wait_reads(b):
            pltpu.make_async_copy(x_hbm.at[0], x_vs[b], in_sems[b]).wait()
            pltpu.make_async_copy(pm_hbm.at[0], p_vs[b], in_sems[b]).wait()
            pltpu.make_async_copy(x_hbm.at[idx_v.at[0]], swap_vs[b],
                                  g_sems[b]).wait()

        def wait_out(b):
            pltpu.make_async_copy(out_vs[b], out_hbm.at[0], o_sems[b]).wait()

        for b in range(nbuf):
            issue_reads(b, b)

        def pair(i, carry):
            for b in range(nbuf):
                gg = i * nbuf + b
                row = bbase + gg
                wait_reads(b)

                @pl.when(gg >= nbuf)
                def _():
                    wait_out(b)

                def sub(r, carry2):
                    pvec = p_vs[b][0, pl.ds(pl.multiple_of(16 * r, 16), 16)]
                    for c in range(32):
                        pr = 8 * r + c // 4
                        off = (c % 4) * 16
                        xc = x_vs[b][pr, pl.ds(off, 16)]
                        sc = swap_vs[b][0, pr, pl.ds(off, 16)]
                        m = (pvec & jnp.uint32(1 << c)) != 0
                        out_vs[b][pr, pl.ds(off, 16)] = jnp.where(
                            m & (xc != 0.0), sc, xc)
                    return carry2

                lax.fori_loop(0, 25, sub, 0)
                pltpu.async_copy(out_vs[b], out_hbm.at[row], o_sems[b])

                @pl.when(gg + nbuf < _B_PER_WORKER)
                def _():
                    issue_reads(b, gg + nbuf)
            return carry

        lax.fori_loop(0, _B_PER_WORKER // nbuf, pair, 0)
        for b in range(nbuf):
            wait_out(b)

    return _swap_kernel


def kernel(inputs):
    out = _make_swap_kernel()(inputs, jnp.asarray(_PMASK),
                              jnp.asarray(_PERM))
    return out


# speedup vs baseline: 1.2864x; 1.2864x over previous
import functools

import jax
import jax.numpy as jnp
import numpy as np
from jax import lax
from jax.experimental import pallas as pl
from jax.experimental.pallas import tpu as pltpu
from jax.experimental.pallas import tpu_sc as plsc

_B, _S, _D = 4096, 200, 64
_SPB = _S // 8
_NSLAB = _B * _SPB
_NC, _NS = 2, 16
_NW = _NC * _NS
_SLAB_PER_WORKER = _NSLAB // _NW
_G = 8
_NGROUPS = _SLAB_PER_WORKER // _G
_NGRP_TOT = _NSLAB // _G


def _tables():
    key = jax.random.key(42)
    k_mask, k_perm = jax.random.split(key)
    mask = np.asarray(jax.random.bernoulli(k_mask, 0.1, (_B, _S, _D)))
    perm = np.asarray(jax.random.permutation(k_perm, _B)).astype(np.int32)
    m = mask.reshape(_NSLAB, 32, 16).astype(np.uint32)
    packed = (m << np.arange(32, dtype=np.uint32)[None, :, None]).sum(
        axis=1, dtype=np.uint32)
    pm = packed.reshape(_NGRP_TOT, 1, 128)
    sidx = (perm[:, None] * _SPB
            + np.arange(_SPB, dtype=np.int32)[None, :])
    return pm, sidx.reshape(_NSLAB).astype(np.int32)


_PMASK, _SIDX = _tables()


@functools.cache
def _make_swap_kernel():
    mesh = plsc.VectorSubcoreMesh(core_axis_name="c", subcore_axis_name="s")
    nbuf = 2

    @functools.partial(
        pl.kernel,
        mesh=mesh,
        out_type=jax.ShapeDtypeStruct((_NSLAB, 8, _D), jnp.float32),
        scratch_types=[
            pltpu.VMEM((_SLAB_PER_WORKER,), jnp.int32),
            *[pltpu.VMEM((_G, 8, _D), jnp.float32) for _ in range(nbuf)],
            *[pltpu.VMEM((_G, 512), jnp.float32) for _ in range(nbuf)],
            *[pltpu.VMEM((_G, 8, _D), jnp.float32) for _ in range(nbuf)],
            *[pltpu.VMEM((1, 128), jnp.uint32) for _ in range(nbuf)],
            *[pltpu.SemaphoreType.DMA for _ in range(3 * nbuf)],
        ],
    )
    def _swap_kernel(x_hbm, x512_hbm, pm_hbm, sidx_hbm, out_hbm, idx_v,
                     x_v0, x_v1, swap_v0, swap_v1, out_v0, out_v1,
                     p_v0, p_v1, in_s0, in_s1, g_s0, g_s1, o_s0, o_s1):
        x_vs, swap_vs, out_vs, p_vs = ((x_v0, x_v1), (swap_v0, swap_v1),
                                       (out_v0, out_v1), (p_v0, p_v1))
        in_sems, g_sems, o_sems = (in_s0, in_s1), (g_s0, g_s1), (o_s0, o_s1)
        wid = lax.axis_index("s") * _NC + lax.axis_index("c")
        sbase = wid * _SLAB_PER_WORKER
        gbase = wid * _NGROUPS
        pltpu.sync_copy(sidx_hbm.at[pl.ds(sbase, _SLAB_PER_WORKER)], idx_v)

        def issue_reads(b, gg):
            slab0 = pl.multiple_of(sbase + gg * _G, _G)
            goff = pl.multiple_of(gg * _G, _G)
            pltpu.async_copy(x_hbm.at[pl.ds(slab0, _G)], x_vs[b], in_sems[b])
            pltpu.async_copy(pm_hbm.at[gbase + gg], p_vs[b], in_sems[b])
            pltpu.async_copy(x512_hbm.at[idx_v.at[pl.ds(goff, _G)]],
                             swap_vs[b], g_sems[b])

        def wait_reads(b):
            pltpu.make_async_copy(
                x_hbm.at[pl.ds(0, _G)], x_vs[b], in_sems[b]).wait()
            pltpu.make_async_copy(pm_hbm.at[0], p_vs[b], in_sems[b]).wait()
            pltpu.make_async_copy(
                x512_hbm.at[idx_v.at[pl.ds(0, _G)]], swap_vs[b],
                g_sems[b]).wait()

        def wait_out(b):
            pltpu.make_async_copy(
                out_vs[b], out_hbm.at[pl.ds(0, _G)], o_sems[b]).wait()

        for b in range(nbuf):
            issue_reads(b, b)

        def pair(i, carry):
            for b in range(nbuf):
                gg = i * nbuf + b
                slab0 = pl.multiple_of(sbase + gg * _G, _G)
                wait_reads(b)

                @pl.when(gg >= nbuf)
                def _():
                    wait_out(b)

                def sub(r, carry2):
                    pvec = p_vs[b][0, pl.ds(pl.multiple_of(16 * r, 16), 16)]
                    for c in range(32):
                        sr = c // 4
                        off = (c % 4) * 16
                        xc = x_vs[b][r, sr, pl.ds(off, 16)]
                        sc = swap_vs[b][r, pl.ds(c * 16, 16)]
                        m = (pvec & jnp.uint32(1 << c)) != 0
                        out_vs[b][r, sr, pl.ds(off, 16)] = jnp.where(
                            m & (xc != 0.0), sc, xc)
                    return carry2

                lax.fori_loop(0, _G, sub, 0)
                pltpu.async_copy(out_vs[b], out_hbm.at[pl.ds(slab0, _G)],
                                 o_sems[b])

                @pl.when(gg + nbuf < _NGROUPS)
                def _():
                    issue_reads(b, gg + nbuf)
            return carry

        lax.fori_loop(0, _NGROUPS // nbuf, pair, 0)
        for b in range(nbuf):
            wait_out(b)

    return _swap_kernel


def kernel(inputs):
    x = inputs.reshape(_NSLAB, 8, _D)
    x512 = inputs.reshape(_NSLAB, 512)
    out = _make_swap_kernel()(x, x512, jnp.asarray(_PMASK),
                              jnp.asarray(_SIDX))
    return out.reshape(_B, _S, _D)
